# baseline (speedup 1.0000x reference)
import functools

import jax
import jax.numpy as jnp
from jax import lax
from jax.experimental import pallas as pl
from jax.experimental.pallas import tpu as pltpu
from jax.experimental.pallas import tpu_sc as plsc

_TOPK = 64
_ROWS = 4096
_COLS = 16384
_NC = 2
_NS = 16
_NW = _NC * _NS
_RPW = _ROWS // _NW
_SAMPLE = 2048
_SBITS = 12
_CPAD = _COLS + 16
_MROWS = 128
_IMIN = -(2 ** 31)


def _key16(v):
    u = plsc.bitcast(v, jnp.int32)
    return u ^ (jnp.right_shift(u, 31) & jnp.int32(0x7FFFFFFF))


def _count_ge(buf_ref, nvec, t):
    def body(i, acc):
        kv = buf_ref[pl.ds(i * 16, 16)]
        return acc + plsc.all_reduce_population_count(kv >= t)
    acc = lax.fori_loop(0, nvec, body, jnp.zeros((16,), jnp.int32))
    return acc[0]


def _sc_body(x_hbm, out_hbm, row_v, skey_v, cand_v, thr_v):
    wid = lax.axis_index("s") * _NC + lax.axis_index("c")
    lane = lax.iota(jnp.int32, 16)
    nsv = _SAMPLE // 16

    def row_work(r):
        pltpu.sync_copy(x_hbm.at[r], row_v)

        def sk(i, _):
            skey_v[pl.ds(i * 16, 16)] = _key16(row_v[pl.ds(i * 16, 16)])
            return 0
        lax.fori_loop(0, nsv, sk, 0)

        cpos = _count_ge(skey_v, nsv, jnp.int32(0))
        t12 = jnp.where(cpos >= _TOPK, jnp.int32(0), _IMIN)
        for b in range(30, 30 - _SBITS, -1):
            cand = t12 | jnp.int32(1 << b)
            c = _count_ge(skey_v, nsv, cand)
            t12 = jnp.where(c >= _TOPK, cand, t12)

        def fb(i, off):
            kv = _key16(row_v[pl.ds(i * 16, 16)])
            m = kv >= t12
            plsc.store_compressed(cand_v.at[pl.ds(off, 16)], kv, mask=m)
            return off + plsc.all_reduce_population_count(m)[0]
        c = lax.fori_loop(0, _COLS // 16, fb, jnp.int32(0))

        cand_v[pl.ds(c, 16)] = jnp.full((16,), _IMIN, jnp.int32)
        nv = (c + 15) // 16
        cpos = _count_ge(cand_v, nv, jnp.int32(0))
        t = jnp.where(cpos >= _TOPK, jnp.int32(0), _IMIN)

        def bit_body(j, tt):
            cand = tt | (jnp.int32(1) << (30 - j))
            cc = _count_ge(cand_v, nv, cand)
            return jnp.where(cc >= _TOPK, cand, tt)
        return lax.fori_loop(0, 31, bit_body, t)

    def group_body(g, _):
        def row_body(j, tvec):
            r = wid * _RPW + g * 16 + j
            t = row_work(r)
            return jnp.where(lane == j, t, tvec)
        tvec = lax.fori_loop(0, 16, row_body, jnp.zeros((16,), jnp.int32))
        thr_v[pl.ds(g * 16, 16)] = tvec
        return 0

    lax.fori_loop(0, _RPW // 16, group_body, 0)
    pltpu.sync_copy(thr_v, out_hbm.at[pl.ds(wid * _RPW, _RPW)])


_sc_thresholds = functools.partial(
    pl.kernel,
    mesh=plsc.VectorSubcoreMesh(core_axis_name="c", subcore_axis_name="s"),
    out_type=jax.ShapeDtypeStruct((_ROWS,), jnp.int32),
    scratch_types=[
        pltpu.VMEM((_COLS,), jnp.float32),
        pltpu.VMEM((_SAMPLE,), jnp.int32),
        pltpu.VMEM((_CPAD,), jnp.int32),
        pltpu.VMEM((_RPW,), jnp.int32),
    ],
    compiler_params=pltpu.CompilerParams(needs_layout_passes=False),
)(_sc_body)


def _mask_block(x_ref, t_ref, o_ref):
    x = x_ref[...]
    tau = t_ref[...]
    o_ref[...] = jnp.where(x >= tau, x, jnp.float32(0.0))


def kernel(inputs):
    x = inputs
    tk = _sc_thresholds(x)
    bits = jnp.where(tk >= 0, tk, tk ^ jnp.int32(0x7FFFFFFF))
    tau = lax.bitcast_convert_type(bits, jnp.float32).reshape(_ROWS, 1)
    return pl.pallas_call(
        _mask_block,
        grid=(_ROWS // _MROWS,),
        in_specs=[
            pl.BlockSpec((_MROWS, _COLS), lambda i: (i, 0)),
            pl.BlockSpec((_MROWS, 1), lambda i: (i, 0)),
        ],
        out_specs=pl.BlockSpec((_MROWS, _COLS), lambda i: (i, 0)),
        out_shape=jax.ShapeDtypeStruct((_ROWS, _COLS), jnp.float32),
        compiler_params=pltpu.CompilerParams(
            dimension_semantics=("arbitrary",)),
    )(x, tau)


# speedup vs baseline: 2.0138x; 2.0138x over previous
import functools

import jax
import jax.numpy as jnp
from jax import lax
from jax.experimental import pallas as pl
from jax.experimental.pallas import tpu as pltpu
from jax.experimental.pallas import tpu_sc as plsc

_TOPK = 64
_ROWS = 4096
_COLS = 16384
_NC = 2
_NS = 16
_NW = _NC * _NS
_RPW = _ROWS // _NW
_MARGIN = 1 << 20
_CPAD = _COLS + 64
_MROWS = 128
_IMIN = -(2 ** 31)


def _key16(v):
    u = plsc.bitcast(v, jnp.int32)
    return u ^ (jnp.right_shift(u, 31) & jnp.int32(0x7FFFFFFF))


def _count_ge(buf_ref, n4, t):
    def body(i, acc):
        base = i * 64
        for u in range(4):
            kv = buf_ref[pl.ds(base + u * 16, 16)]
            acc = acc + plsc.all_reduce_population_count(kv >= t)
        return acc
    acc = lax.fori_loop(0, n4, body, jnp.zeros((16,), jnp.int32))
    return acc[0]


def _sc_body(x_hbm, out_hbm, row0_v, row1_v, cand_v, thr_v, sem):
    wid = lax.axis_index("s") * _NC + lax.axis_index("c")
    lane = lax.iota(jnp.int32, 16)
    r0 = wid * _RPW

    pltpu.async_copy(x_hbm.at[r0], row0_v, sem)

    def filter_pass(row_ref, t0):
        def fb(i, off):
            base = i * 64
            for u in range(4):
                kv = _key16(row_ref[pl.ds(base + u * 16, 16)])
                m = kv >= t0
                plsc.store_compressed(cand_v.at[pl.ds(off, 16)], kv, mask=m)
                off = off + plsc.all_reduce_population_count(m)[0]
            return off
        return lax.fori_loop(0, _COLS // 64, fb, jnp.int32(0))

    def select_threshold(row_ref, t_prev):
        t0 = t_prev - jnp.int32(_MARGIN)
        t0 = jnp.where(t0 > t_prev, jnp.int32(_IMIN), t0)
        c = filter_pass(row_ref, t0)

        def wcond(carry):
            return carry < _TOPK

        def wbody(carry):
            return filter_pass(row_ref, jnp.int32(_IMIN))

        c = lax.while_loop(wcond, wbody, c)

        pad = jnp.full((16,), _IMIN, jnp.int32)
        for u in range(4):
            cand_v[pl.ds(c + u * 16, 16)] = pad
        n4 = (c + 63) // 64
        cpos = _count_ge(cand_v, n4, jnp.int32(0))
        t = jnp.where(cpos >= _TOPK, jnp.int32(0), jnp.int32(_IMIN))

        def bit_body(j, tt):
            cand = tt | (jnp.int32(1) << (30 - j))
            cc = _count_ge(cand_v, n4, cand)
            return jnp.where(cc >= _TOPK, cand, tt)
        return lax.fori_loop(0, 31, bit_body, t)

    def pair_body(p, carry):
        t_prev, tvec = carry
        r = r0 + 2 * p
        pltpu.make_async_copy(x_hbm.at[r], row0_v, sem).wait()
        pltpu.async_copy(x_hbm.at[r + 1], row1_v, sem)
        ta = select_threshold(row0_v, t_prev)
        tvec = jnp.where(lane == (2 * p) % 16, ta, tvec)

        pltpu.make_async_copy(x_hbm.at[r + 1], row1_v, sem).wait()

        @pl.when(2 * p + 2 < _RPW)
        def _():
            pltpu.async_copy(x_hbm.at[r + 2], row0_v, sem)

        tb = select_threshold(row1_v, ta)
        tvec = jnp.where(lane == (2 * p) % 16 + 1, tb, tvec)

        @pl.when((2 * p) % 16 == 14)
        def _():
            thr_v[pl.ds(((2 * p) // 16) * 16, 16)] = tvec

        return tb, tvec

    lax.fori_loop(0, _RPW // 2, pair_body,
                  (jnp.int32(_IMIN), jnp.zeros((16,), jnp.int32)))
    pltpu.sync_copy(thr_v, out_hbm.at[pl.ds(r0, _RPW)])


_sc_thresholds = functools.partial(
    pl.kernel,
    mesh=plsc.VectorSubcoreMesh(core_axis_name="c", subcore_axis_name="s"),
    out_type=jax.ShapeDtypeStruct((_ROWS,), jnp.int32),
    scratch_types=[
        pltpu.VMEM((_COLS,), jnp.float32),
        pltpu.VMEM((_COLS,), jnp.float32),
        pltpu.VMEM((_CPAD,), jnp.int32),
        pltpu.VMEM((_RPW,), jnp.int32),
        pltpu.SemaphoreType.DMA,
    ],
    compiler_params=pltpu.CompilerParams(needs_layout_passes=False),
)(_sc_body)


def _mask_block(x_ref, t_ref, o_ref):
    x = x_ref[...]
    tau = t_ref[...]
    o_ref[...] = jnp.where(x >= tau, x, jnp.float32(0.0))


def kernel(inputs):
    x = inputs
    tk = _sc_thresholds(x)
    bits = jnp.where(tk >= 0, tk, tk ^ jnp.int32(0x7FFFFFFF))
    tau = lax.bitcast_convert_type(bits, jnp.float32).reshape(_ROWS, 1)
    return pl.pallas_call(
        _mask_block,
        grid=(_ROWS // _MROWS,),
        in_specs=[
            pl.BlockSpec((_MROWS, _COLS), lambda i: (i, 0)),
            pl.BlockSpec((_MROWS, 1), lambda i: (i, 0)),
        ],
        out_specs=pl.BlockSpec((_MROWS, _COLS), lambda i: (i, 0)),
        out_shape=jax.ShapeDtypeStruct((_ROWS, _COLS), jnp.float32),
        compiler_params=pltpu.CompilerParams(
            dimension_semantics=("arbitrary",)),
    )(x, tau)
